# baseline (speedup 1.0000x reference)
import functools

import jax
import jax.numpy as jnp
from jax import lax
from jax.experimental import pallas as pl
from jax.experimental.pallas import tpu as pltpu
from jax.experimental.pallas import tpu_sc as plsc

EMBED_DIM = 128
NC, NS = 2, 16
NW = NC * NS
CHUNK = 128
GROUP = 2
BANK = GROUP * CHUNK


def _concat_body(tok_ref, cul_ref, out_ref):
    out_ref[...] = jnp.concatenate(
        [tok_ref[...].T, cul_ref[...].T], axis=1)


@functools.cache
def _make_concat(v: int, d_tok: int, d_cul: int):
    cols = 2048
    grid = (v + cols - 1) // cols
    return pl.pallas_call(
        _concat_body,
        grid=(grid,),
        in_specs=[
            pl.BlockSpec((d_tok, cols), lambda i: (0, i)),
            pl.BlockSpec((d_cul, cols), lambda i: (0, i)),
        ],
        out_specs=pl.BlockSpec((cols, d_tok + d_cul), lambda i: (i, 0)),
        out_shape=jax.ShapeDtypeStruct((v, d_tok + d_cul), jnp.float32),
    )


@functools.cache
def _make_gather(n_tokens: int):
    per_w = n_tokens // NW
    n_bank = per_w // BANK
    n_round = (n_bank - 1) // 3
    mesh = plsc.VectorSubcoreMesh(
        core_axis_name="c", subcore_axis_name="s",
        num_cores=NC, num_subcores=NS)

    @functools.partial(
        pl.kernel,
        out_type=jax.ShapeDtypeStruct((n_tokens, EMBED_DIM), jnp.float32),
        mesh=mesh,
        scratch_types=[
            pltpu.VMEM((per_w,), jnp.int32),
            pltpu.VMEM((BANK, EMBED_DIM), jnp.float32),
            pltpu.VMEM((BANK, EMBED_DIM), jnp.float32),
            pltpu.VMEM((BANK, EMBED_DIM), jnp.float32),
            pltpu.SemaphoreType.DMA,
            pltpu.SemaphoreType.DMA,
            pltpu.SemaphoreType.DMA,
            pltpu.SemaphoreType.DMA,
            pltpu.SemaphoreType.DMA,
            pltpu.SemaphoreType.DMA,
        ],
    )
    def k(ids_hbm, cat_hbm, out_hbm, idx_v,
          buf0, buf1, buf2, g0, g1, g2, w0, w1, w2):
        wid = lax.axis_index("s") * NC + lax.axis_index("c")
        base = wid * per_w
        pltpu.sync_copy(ids_hbm.at[pl.ds(base, per_w)], idx_v)

        def gather(bank, buf, sem):
            for c in range(GROUP):
                pltpu.async_copy(
                    cat_hbm.at[idx_v.at[pl.ds(bank * BANK + c * CHUNK, CHUNK)]],
                    buf.at[pl.ds(c * CHUNK, CHUNK)], sem)

        def write(bank, buf, sem):
            pltpu.async_copy(
                buf, out_hbm.at[pl.ds(base + bank * BANK, BANK)], sem)

        def gwait(buf, sem):
            for c in range(GROUP):
                pltpu.make_async_copy(
                    cat_hbm.at[idx_v.at[pl.ds(0, CHUNK)]],
                    buf.at[pl.ds(c * CHUNK, CHUNK)], sem).wait()

        def wwait(buf, sem):
            pltpu.make_async_copy(
                buf, out_hbm.at[pl.ds(base, BANK)], sem).wait()

        gather(0, buf0, g0)
        gather(1, buf1, g1)

        @pl.loop(0, n_round)
        def _(i):
            t0 = 3 * i
            gwait(buf0, g0)

            @pl.when(i > 0)
            def _():
                wwait(buf2, w2)
            gather(t0 + 2, buf2, g2)
            write(t0, buf0, w0)
            gwait(buf1, g1)
            write(t0 + 1, buf1, w1)
            wwait(buf0, w0)
            gather(t0 + 3, buf0, g0)
            gwait(buf2, g2)
            write(t0 + 2, buf2, w2)
            wwait(buf1, w1)

            @pl.when(t0 + 4 < n_bank)
            def _():
                gather(t0 + 4, buf1, g1)

        gwait(buf0, g0)
        wwait(buf2, w2)
        write(n_bank - 1, buf0, w0)
        wwait(buf0, w0)

    return k


def kernel(input_ids, token_weight, cultural_weight):
    b, s = input_ids.shape
    n = b * s
    v, d_tok = token_weight.shape
    d_cul = cultural_weight.shape[1]
    ids_flat = input_ids.T.astype(jnp.int32).reshape(n)
    cat = _make_concat(v, d_tok, d_cul)(token_weight.T, cultural_weight.T)
    out = _make_gather(n)(ids_flat, cat)
    return out.reshape(s, b, EMBED_DIM).transpose(1, 0, 2)
